# baseline (speedup 1.0000x reference)
import functools

import jax
import jax.numpy as jnp
from jax import lax
from jax.experimental import pallas as pl
from jax.experimental.pallas import tpu as pltpu
from jax.experimental.pallas import tpu_sc as plsc

N_NODES = 10000
D = 128
NC = 2
NS = 16
NW = NC * NS
K = 128
L = 16

SP_ROWS = 10240


def _split(e):
    t = -(-e // K)
    per = -(-t // NS)
    ch0 = max(8, int(round(per * 0.50 / 8)) * 8)
    ch1 = max(8, -(-(per - ch0) // 8) * 8)
    return ch0, ch1


def _sc_segment_sum(x, src3, dst3, ch0, ch1):
    mesh = plsc.VectorSubcoreMesh(core_axis_name="c", subcore_axis_name="s",
                                  num_cores=NC, num_subcores=NS)

    hc = ch0 // 2

    @functools.partial(
        pl.kernel,
        out_type=jax.ShapeDtypeStruct((NC, N_NODES, D), jnp.float32),
        mesh=mesh,
        scratch_types=[
            pltpu.VMEM((hc, K), jnp.int32),
            pltpu.VMEM((hc, K), jnp.int32),
            pltpu.VMEM((K, D), jnp.float32),
            pltpu.VMEM((K, D), jnp.float32),
            pltpu.SemaphoreType.DMA,
            pltpu.SemaphoreType.DMA,
            pltpu.VMEM_SHARED((SP_ROWS, D), jnp.float32),
        ],
    )
    def k(x_hbm, src_hbm, dst_hbm, out_hbm, src_v, dst_v, rows0, rows1,
          sem0, sem1, acc_sp):
        bufs = (rows0, rows1)
        sems = (sem0, sem1)
        c = lax.axis_index("c")
        s = lax.axis_index("s")
        wid = c * NS + s

        zeros = jnp.zeros((L,), jnp.float32)

        def zbody(i, _):
            r = i // (D // L)
            col = (i % (D // L)) * L
            rows1[r, pl.ds(col, L)] = zeros
            return 0

        def zero_acc():
            lax.fori_loop(0, K * (D // L), zbody, 0)
            for j in range(SP_ROWS // (NS * K)):
                pltpu.sync_copy(
                    rows1, acc_sp.at[pl.ds(s * (SP_ROWS // NS) + j * K, K)])
            plsc.subcore_barrier()

        def gwait(b):
            pltpu.make_async_copy(x_hbm.at[pl.ds(0, K)], bufs[b],
                                  sems[b]).wait()

        with jax.named_scope("edge_loop"):
            for h in range(2):
                pltpu.sync_copy(src_hbm.at[wid, pl.ds(h * hc, hc)], src_v)
                pltpu.sync_copy(dst_hbm.at[wid, pl.ds(h * hc, hc)], dst_v)
                pltpu.async_copy(x_hbm.at[src_v.at[0]], bufs[0], sems[0])
                if h == 0:
                    zero_acc()

                def group(g, _):
                    j = g * 2
                    pltpu.async_copy(x_hbm.at[src_v.at[j + 1]], bufs[1],
                                     sems[1])
                    gwait(0)
                    pltpu.sync_copy(bufs[0], acc_sp.at[dst_v.at[j]], add=True)
                    pltpu.async_copy(x_hbm.at[src_v.at[j + 2]], bufs[0],
                                     sems[0])
                    gwait(1)
                    pltpu.sync_copy(bufs[1], acc_sp.at[dst_v.at[j + 1]],
                                    add=True)
                    return 0

                lax.fori_loop(0, hc // 2 - 1, group, 0)
                j = hc - 2
                pltpu.async_copy(x_hbm.at[src_v.at[j + 1]], bufs[1], sems[1])
                gwait(0)
                pltpu.sync_copy(bufs[0], acc_sp.at[dst_v.at[j]], add=True)
                gwait(1)
                pltpu.sync_copy(bufs[1], acc_sp.at[dst_v.at[j + 1]], add=True)
            plsc.subcore_barrier()

        with jax.named_scope("writeout"):
            rpt = (N_NODES // NS) // 8 * 8
            base = s * rpt
            pltpu.sync_copy(acc_sp.at[pl.ds(base, rpt)],
                            out_hbm.at[c, pl.ds(base, rpt)])

            @pl.when(s == NS - 1)
            def _tail():
                t0 = NS * rpt
                pltpu.sync_copy(acc_sp.at[pl.ds(t0, N_NODES - t0)],
                                out_hbm.at[c, pl.ds(t0, N_NODES - t0)])

    return k(x, src3, dst3)


def _tc_root(x, w_root_t):
    bn = 2000

    def body(x_ref, wo_ref, o_ref):
        o_ref[...] = jnp.dot(x_ref[...], wo_ref[...],
                             preferred_element_type=jnp.float32)

    return pl.pallas_call(
        body,
        grid=(N_NODES // bn,),
        in_specs=[
            pl.BlockSpec((bn, D), lambda i: (i, 0)),
            pl.BlockSpec((D, D), lambda i: (0, 0)),
        ],
        out_specs=pl.BlockSpec((bn, D), lambda i: (i, 0)),
        out_shape=jax.ShapeDtypeStruct((N_NODES, D), jnp.float32),
    )(x, w_root_t)


def _tc_combine(partials, root, w_rel_t):
    bn = 2000

    def body(p_ref, r_ref, wr_ref, o_ref):
        agg = p_ref[0] + p_ref[1]
        o_ref[...] = (
            jnp.dot(agg, wr_ref[...], preferred_element_type=jnp.float32)
            + r_ref[...]
        )

    return pl.pallas_call(
        body,
        grid=(N_NODES // bn,),
        in_specs=[
            pl.BlockSpec((NC, bn, D), lambda i: (0, i, 0)),
            pl.BlockSpec((bn, D), lambda i: (i, 0)),
            pl.BlockSpec((D, D), lambda i: (0, 0)),
        ],
        out_specs=pl.BlockSpec((bn, D), lambda i: (i, 0)),
        out_shape=jax.ShapeDtypeStruct((N_NODES, D), jnp.float32),
    )(partials, root, w_rel_t)


def kernel(x, edge_index, W_rel, W_root):
    e = edge_index.shape[1]
    src = edge_index[0].astype(jnp.int32)
    dst = edge_index[1].astype(jnp.int32)
    ch0, ch1 = _split(e)

    e_pad = NS * (ch0 + ch1) * K
    n_pad = e_pad - e
    pad_src = jnp.arange(n_pad, dtype=jnp.int32) % N_NODES
    pad_dst = N_NODES + jnp.arange(n_pad, dtype=jnp.int32) % (SP_ROWS - N_NODES)
    src = jnp.concatenate([src, pad_src]).reshape(-1, K)
    dst = jnp.concatenate([dst, pad_dst]).reshape(-1, K)

    def per_worker(a):
        p0 = a[:NS * ch0].reshape(NS, ch0, K)
        p1 = a[NS * ch0:].reshape(NS, ch1, K)
        p1 = jnp.pad(p1, ((0, 0), (0, ch0 - ch1), (0, 0)))
        return jnp.concatenate([p0, p1], axis=0)

    root = _tc_root(x, W_root.T)
    partials = _sc_segment_sum(x, per_worker(src), per_worker(dst), ch0, ch1)
    return _tc_combine(partials, root, W_rel.T)


# speedup vs baseline: 1.0204x; 1.0204x over previous
import functools

import jax
import jax.numpy as jnp
from jax import lax
from jax.experimental import pallas as pl
from jax.experimental.pallas import tpu as pltpu
from jax.experimental.pallas import tpu_sc as plsc

N_NODES = 10000
D = 128
NC = 2
NS = 16
NW = NC * NS
K = 128
L = 16

SP_ROWS = 10240


def _split(e):
    t = -(-e // K)
    return -(-t // (NW * 16)) * 16


def _sc_segment_sum(x, src3, dst3, ch):
    mesh = plsc.VectorSubcoreMesh(core_axis_name="c", subcore_axis_name="s",
                                  num_cores=NC, num_subcores=NS)

    hc = ch // 2

    @functools.partial(
        pl.kernel,
        out_type=jax.ShapeDtypeStruct((NC, N_NODES, D), jnp.float32),
        mesh=mesh,
        scratch_types=[
            pltpu.VMEM((hc, K), jnp.int32),
            pltpu.VMEM((hc, K), jnp.int32),
            pltpu.VMEM((K, D), jnp.float32),
            pltpu.VMEM((K, D), jnp.float32),
            pltpu.SemaphoreType.DMA,
            pltpu.SemaphoreType.DMA,
            pltpu.VMEM_SHARED((SP_ROWS, D), jnp.float32),
        ],
    )
    def k(x_hbm, src_hbm, dst_hbm, out_hbm, src_v, dst_v, rows0, rows1,
          sem0, sem1, acc_sp):
        bufs = (rows0, rows1)
        sems = (sem0, sem1)
        c = lax.axis_index("c")
        s = lax.axis_index("s")
        wid = c * NS + s

        zeros = jnp.zeros((L,), jnp.float32)

        def zbody(i, _):
            r = i // (D // L)
            col = (i % (D // L)) * L
            rows1[r, pl.ds(col, L)] = zeros
            return 0

        def zero_acc():
            lax.fori_loop(0, K * (D // L), zbody, 0)
            for j in range(SP_ROWS // (NS * K)):
                pltpu.sync_copy(
                    rows1, acc_sp.at[pl.ds(s * (SP_ROWS // NS) + j * K, K)])
            plsc.subcore_barrier()

        def gwait(b):
            pltpu.make_async_copy(x_hbm.at[pl.ds(0, K)], bufs[b],
                                  sems[b]).wait()

        with jax.named_scope("edge_loop"):
            for h in range(2):
                pltpu.sync_copy(src_hbm.at[wid, pl.ds(h * hc, hc)], src_v)
                pltpu.sync_copy(dst_hbm.at[wid, pl.ds(h * hc, hc)], dst_v)
                pltpu.async_copy(x_hbm.at[src_v.at[0]], bufs[0], sems[0])
                if h == 0:
                    zero_acc()

                def group(g, _):
                    j = g * 2
                    pltpu.async_copy(x_hbm.at[src_v.at[j + 1]], bufs[1],
                                     sems[1])
                    gwait(0)
                    pltpu.sync_copy(bufs[0], acc_sp.at[dst_v.at[j]], add=True)
                    pltpu.async_copy(x_hbm.at[src_v.at[j + 2]], bufs[0],
                                     sems[0])
                    gwait(1)
                    pltpu.sync_copy(bufs[1], acc_sp.at[dst_v.at[j + 1]],
                                    add=True)
                    return 0

                lax.fori_loop(0, hc // 2 - 1, group, 0)
                j = hc - 2
                pltpu.async_copy(x_hbm.at[src_v.at[j + 1]], bufs[1], sems[1])
                gwait(0)
                pltpu.sync_copy(bufs[0], acc_sp.at[dst_v.at[j]], add=True)
                gwait(1)
                pltpu.sync_copy(bufs[1], acc_sp.at[dst_v.at[j + 1]], add=True)
            plsc.subcore_barrier()

        with jax.named_scope("writeout"):
            rpt = (N_NODES // NS) // 8 * 8
            base = s * rpt
            pltpu.sync_copy(acc_sp.at[pl.ds(base, rpt)],
                            out_hbm.at[c, pl.ds(base, rpt)])

            @pl.when(s == NS - 1)
            def _tail():
                t0 = NS * rpt
                pltpu.sync_copy(acc_sp.at[pl.ds(t0, N_NODES - t0)],
                                out_hbm.at[c, pl.ds(t0, N_NODES - t0)])

    return k(x, src3, dst3)


def _tc_root(x, w_root_t):
    bn = 2000

    def body(x_ref, wo_ref, o_ref):
        o_ref[...] = jnp.dot(x_ref[...], wo_ref[...],
                             preferred_element_type=jnp.float32)

    return pl.pallas_call(
        body,
        grid=(N_NODES // bn,),
        in_specs=[
            pl.BlockSpec((bn, D), lambda i: (i, 0)),
            pl.BlockSpec((D, D), lambda i: (0, 0)),
        ],
        out_specs=pl.BlockSpec((bn, D), lambda i: (i, 0)),
        out_shape=jax.ShapeDtypeStruct((N_NODES, D), jnp.float32),
    )(x, w_root_t)


def _tc_combine(partials, root, w_rel_t):
    bn = 2000

    def body(p_ref, r_ref, wr_ref, o_ref):
        agg = p_ref[0] + p_ref[1]
        o_ref[...] = (
            jnp.dot(agg, wr_ref[...], preferred_element_type=jnp.float32)
            + r_ref[...]
        )

    return pl.pallas_call(
        body,
        grid=(N_NODES // bn,),
        in_specs=[
            pl.BlockSpec((NC, bn, D), lambda i: (0, i, 0)),
            pl.BlockSpec((bn, D), lambda i: (i, 0)),
            pl.BlockSpec((D, D), lambda i: (0, 0)),
        ],
        out_specs=pl.BlockSpec((bn, D), lambda i: (i, 0)),
        out_shape=jax.ShapeDtypeStruct((N_NODES, D), jnp.float32),
    )(partials, root, w_rel_t)


def kernel(x, edge_index, W_rel, W_root):
    e = edge_index.shape[1]
    src = edge_index[0].astype(jnp.int32)
    dst = edge_index[1].astype(jnp.int32)
    ch = _split(e)

    e_pad = NW * ch * K
    n_pad = e_pad - e
    pad_src = jnp.arange(n_pad, dtype=jnp.int32) % N_NODES
    pad_dst = N_NODES + jnp.arange(n_pad, dtype=jnp.int32) % (SP_ROWS - N_NODES)
    src = jnp.concatenate([src, pad_src]).reshape(NW, ch, K)
    dst = jnp.concatenate([dst, pad_dst]).reshape(NW, ch, K)

    root = _tc_root(x, W_root.T)
    partials = _sc_segment_sum(x, src, dst, ch)
    return _tc_combine(partials, root, W_rel.T)
